# Initial kernel scaffold; baseline (speedup 1.0000x reference)
import jax
import jax.numpy as jnp
from jax.experimental import pallas as pl


def kernel(X, C, W_node, b_node, W_edge, b_edge, W_msg, b_msg):
    raise NotImplementedError("write your pallas kernel here")


# speedup vs baseline: 7.8767x; 7.8767x over previous
import functools

import jax
import jax.numpy as jnp
from jax import lax
from jax.experimental import pallas as pl
from jax.experimental.pallas import tpu as pltpu
from jax.experimental.pallas import tpu_sc as plsc

B = 4
N = 2048
K = 30
DN = 256
DE = 128
E = B * N * K


_RA = 512


def _sp(x):
    return jnp.maximum(x, 0.0) + jnp.log(1.0 + jnp.exp(-jnp.abs(x)))


def _node_feat_body(nf_ref, c_ref, wn_ref, bn_ref, w1_ref, w2_ref, bm_ref,
                    nh_ref, p1_ref, p2_ref, mi_ref):
    mask = (c_ref[...] > 0).astype(jnp.float32)
    nh = _sp(jnp.dot(nf_ref[...], wn_ref[...],
                     preferred_element_type=jnp.float32) + bn_ref[...])
    nh = nh * mask
    nh_ref[...] = nh
    p1_ref[...] = jnp.dot(nh, w1_ref[...],
                          preferred_element_type=jnp.float32) + bm_ref[...]
    p2_ref[...] = jnp.dot(nh, w2_ref[...],
                          preferred_element_type=jnp.float32)
    mi_ref[...] = mask


def _node_feat(nf, c32, W_node, b_node, W1, W2, b_msg):
    nblk = (B * N) // _RA
    full = lambda s: pl.BlockSpec(s, lambda i: (0, 0))
    return pl.pallas_call(
        _node_feat_body,
        grid=(nblk,),
        in_specs=[
            pl.BlockSpec((_RA, 16), lambda i: (i, 0)),
            pl.BlockSpec((_RA, 1), lambda i: (i, 0)),
            full((16, DN)),
            full((1, DN)),
            full((DN, DN)),
            full((DN, DN)),
            full((1, DN)),
        ],
        out_specs=[
            pl.BlockSpec((_RA, DN), lambda i: (i, 0)),
            pl.BlockSpec((_RA, DN), lambda i: (i, 0)),
            pl.BlockSpec((_RA, DN), lambda i: (i, 0)),
            pl.BlockSpec((_RA, 1), lambda i: (i, 0)),
        ],
        out_shape=[
            jax.ShapeDtypeStruct((B * N, DN), jnp.float32),
            jax.ShapeDtypeStruct((B * N, DN), jnp.float32),
            jax.ShapeDtypeStruct((B * N, DN), jnp.float32),
            jax.ShapeDtypeStruct((B * N, 1), jnp.float32),
        ],
    )(nf, c32, W_node, b_node, W1, W2, b_msg)


_RB = 256


def _topk_body(cxr, cyr, czr, cxc, cyc, czc, mc_ref, cc_ref,
               d_ref, i_ref, f_ref, cj_ref):
    b = pl.program_id(0)
    ib = pl.program_id(1)
    xr = cxr[0]
    yr = cyr[0]
    zr = czr[0]
    xc = cxc[0]
    yc = cyc[0]
    zc = czc[0]
    dx = xr - xc
    dy = yr - yc
    dz = zr - zc
    D = jnp.sqrt(dx * dx + dy * dy + dz * dz + 1e-8)
    col = lax.broadcasted_iota(jnp.int32, (1, N), 1)
    row = ib * _RB + lax.broadcasted_iota(jnp.int32, (_RB, 1), 0)
    D = D + jnp.where(row == col, 1e6, 0.0).astype(jnp.float32)
    D = D + (1.0 - mc_ref[0]) * 1e6
    col4 = col * 4 + cc_ref[0]
    big = jnp.int32(4 * N)
    inf = jnp.float32(jnp.inf)
    ds, js, cs = [], [], []
    for _ in range(K):
        m = jnp.min(D, axis=1, keepdims=True)
        cand = jnp.where(D == m, col4, big)
        j4 = jnp.min(cand, axis=1, keepdims=True)
        j = jnp.right_shift(j4, 2)
        ds.append(m)
        js.append(j)
        cs.append(jnp.bitwise_and(j4, 3))
        D = jnp.where(col == j, inf, D)
    d_ref[0] = jnp.concatenate(ds, axis=1)
    i_out = jnp.concatenate(js, axis=1)
    i_ref[0] = i_out
    f_ref[0] = i_out + b * N
    cj_ref[0] = jnp.concatenate(cs, axis=1)


def _topk(centroid, mask_i, c32):
    cr = [centroid[:, :, a:a + 1] for a in range(3)]
    cc = [jnp.transpose(centroid[:, :, a:a + 1], (0, 2, 1)) for a in range(3)]
    mc = mask_i[:, None, :]
    ccol = c32[:, None, :]
    nblk = N // _RB
    rspec = pl.BlockSpec((1, _RB, 1), lambda b, i: (b, i, 0))
    cspec = pl.BlockSpec((1, 1, N), lambda b, i: (b, 0, 0))
    ospec = pl.BlockSpec((1, _RB, K), lambda b, i: (b, i, 0))
    return pl.pallas_call(
        _topk_body,
        grid=(B, nblk),
        in_specs=[rspec, rspec, rspec, cspec, cspec, cspec, cspec, cspec],
        out_specs=[ospec, ospec, ospec, ospec],
        out_shape=[
            jax.ShapeDtypeStruct((B, N, K), jnp.float32),
            jax.ShapeDtypeStruct((B, N, K), jnp.int32),
            jax.ShapeDtypeStruct((B, N, K), jnp.int32),
            jax.ShapeDtypeStruct((B, N, K), jnp.int32),
        ],
    )(cr[0], cr[1], cr[2], cc[0], cc[1], cc[2], mc, ccol)


_CH = 128


def _sc_gather(p2t, eidx):
    info = plsc.get_sparse_core_info()
    nw = info.num_cores * info.num_subcores
    per_w = E // nw
    nch = per_w // _CH
    mesh = plsc.VectorSubcoreMesh(core_axis_name="c", subcore_axis_name="s")

    @functools.partial(
        pl.kernel,
        mesh=mesh,
        out_type=jax.ShapeDtypeStruct((E, DN), jnp.float32),
        scratch_types=[
            pltpu.VMEM((_CH,), jnp.int32),
            pltpu.VMEM((_CH,), jnp.int32),
            pltpu.VMEM((_CH, DN), jnp.float32),
            pltpu.VMEM((_CH, DN), jnp.float32),
            pltpu.SemaphoreType.DMA,
            pltpu.SemaphoreType.DMA,
        ],
    )
    def k(p2_hbm, eidx_hbm, g_hbm, idxa, idxb, rowa, rowb, sema, semb):
        wid = lax.axis_index("s") * info.num_cores + lax.axis_index("c")
        base = wid * per_w

        def pair(t, idxv, rowv, sem):
            off = base + t * _CH
            pltpu.sync_copy(eidx_hbm.at[pl.ds(off, _CH)], idxv)
            pltpu.async_copy(p2_hbm.at[idxv], rowv, sem).wait()
            pltpu.sync_copy(rowv, g_hbm.at[pl.ds(off, _CH)])

        def body(t2, carry):
            pair(t2 * 2, idxa, rowa, sema)
            pair(t2 * 2 + 1, idxb, rowb, semb)
            return carry

        lax.fori_loop(0, nch // 2, body, 0)

    return k(p2t, eidx)


_RD = 64
_ED = _RD * K

_SIGMA = 20.0 / 16.0


def _edge_msg_body(d1, cjr, crep, mrep, mi, nh, p1, g,
                   we_ref, be_ref, w3_ref, cen_ref,
                   eh_ref, mij_ref, no_ref):
    inv = jnp.float32(1.0 / (2.0 * _SIGMA * _SIGMA))
    dd = d1[...] - cen_ref[...]
    rbf = jnp.exp(-(dd * dd) * inv)
    cj = cjr[...]
    same = (crep[...] == cj).astype(jnp.float32)
    we = we_ref[...]
    pre = (jnp.dot(rbf, we[:16, :], preferred_element_type=jnp.float32)
           + same * we[16:17, :] + be_ref[...])
    m_row = mrep[...] * (cj > 0).astype(jnp.float32)
    eh = _sp(pre) * m_row
    eh_ref[...] = eh
    mij_ref[...] = m_row
    e3 = jnp.dot(eh, w3_ref[...], preferred_element_type=jnp.float32)
    p1b = jnp.reshape(
        jnp.broadcast_to(p1[...][:, None, :], (_RD, K, DN)), (_ED, DN))
    msg = _sp(p1b + g[...] + e3) * m_row
    msum = jnp.sum(jnp.reshape(msg, (_RD, K, DN)), axis=1)
    denom = jnp.sum(jnp.reshape(m_row, (_RD, K, 1)), axis=1) + 1e-6
    no_ref[...] = (nh[...] + msum / denom) * mi[...]


def _edge_msg(d1, cjr, crep, mrep, mi, nh, p1, g, W_edge, b_edge, W3):
    nblk = (B * N) // _RD
    rs = lambda w: pl.BlockSpec((_RD, w), lambda i: (i, 0))
    es = lambda w: pl.BlockSpec((_ED, w), lambda i: (i, 0))
    full = lambda s: pl.BlockSpec(s, lambda i: (0, 0))
    centers = jnp.linspace(0.0, 20.0, 16).astype(jnp.float32).reshape(1, 16)
    return pl.pallas_call(
        _edge_msg_body,
        grid=(nblk,),
        in_specs=[
            es(1),
            es(1),
            es(1),
            es(1),
            rs(1),
            rs(DN),
            rs(DN),
            es(DN),
            full((17, DE)),
            full((1, DE)),
            full((DE, DN)),
            full((1, 16)),
        ],
        out_specs=[es(DE), es(1), rs(DN)],
        out_shape=[
            jax.ShapeDtypeStruct((E, DE), jnp.float32),
            jax.ShapeDtypeStruct((E, 1), jnp.float32),
            jax.ShapeDtypeStruct((B * N, DN), jnp.float32),
        ],
    )(d1, cjr, crep, mrep, mi, nh, p1, g, W_edge, b_edge, W3, centers)


def kernel(X, C, W_node, b_node, W_edge, b_edge, W_msg, b_msg):
    c32 = C.astype(jnp.int32)
    nf = X.reshape(B, N, 12).reshape(B * N, 12)
    nf = jnp.pad(nf, ((0, 0), (0, 4)))
    Wn = jnp.pad(W_node, ((0, 4), (0, 0)))
    W1 = W_msg[0:DN]
    W2 = W_msg[DN:2 * DN]
    W3 = W_msg[2 * DN:]

    nh, p1, p2, mi_col = _node_feat(
        nf, c32.reshape(B * N, 1), Wn, b_node.reshape(1, DN),
        W1, W2, b_msg.reshape(1, DN))

    centroid = X.mean(axis=2)
    mask_i = mi_col.reshape(B, N)
    d_ij, edge_idx, eidx_flat, cj = _topk(centroid, mask_i, c32)

    g = _sc_gather(p2, eidx_flat.reshape(E))

    d1 = d_ij.reshape(E, 1)
    cjr = cj.reshape(E, 1)
    crep = jnp.broadcast_to(
        c32.reshape(B * N, 1, 1), (B * N, K, 1)).reshape(E, 1)
    mrep = jnp.broadcast_to(
        mi_col.reshape(B * N, 1, 1), (B * N, K, 1)).reshape(E, 1)

    eh, mij, no = _edge_msg(d1, cjr, crep, mrep, mi_col, nh, p1, g,
                            W_edge, b_edge.reshape(1, DE), W3)

    return (no.reshape(B, N, DN), eh.reshape(B, N, K, DE), edge_idx,
            mask_i, mij.reshape(B, N, K))


# speedup vs baseline: 8.4404x; 1.0716x over previous
import functools

import jax
import jax.numpy as jnp
from jax import lax
from jax.experimental import pallas as pl
from jax.experimental.pallas import tpu as pltpu
from jax.experimental.pallas import tpu_sc as plsc

B = 4
N = 2048
K = 30
DN = 256
DE = 128
E = B * N * K


_RA = 512


def _sp(x):
    return jnp.maximum(x, 0.0) + jnp.log(1.0 + jnp.exp(-jnp.abs(x)))


def _node_feat_body(nf_ref, c_ref, wn_ref, bn_ref, w1_ref, w2_ref, bm_ref,
                    nh_ref, p1_ref, p2a_ref, p2b_ref, mi_ref):
    mask = (c_ref[...] > 0).astype(jnp.float32)
    nh = _sp(jnp.dot(nf_ref[...], wn_ref[...],
                     preferred_element_type=jnp.float32) + bn_ref[...])
    nh = nh * mask
    nh_ref[...] = nh
    p1_ref[...] = jnp.dot(nh, w1_ref[...],
                          preferred_element_type=jnp.float32) + bm_ref[...]
    p2 = jnp.dot(nh, w2_ref[...], preferred_element_type=jnp.float32)
    p2a_ref[...] = p2[:, :DE]
    p2b_ref[...] = p2[:, DE:]
    mi_ref[...] = mask


def _node_feat(nf, c32, W_node, b_node, W1, W2, b_msg):
    nblk = (B * N) // _RA
    full = lambda s: pl.BlockSpec(s, lambda i: (0, 0))
    return pl.pallas_call(
        _node_feat_body,
        grid=(nblk,),
        in_specs=[
            pl.BlockSpec((_RA, 16), lambda i: (i, 0)),
            pl.BlockSpec((_RA, 1), lambda i: (i, 0)),
            full((16, DN)),
            full((1, DN)),
            full((DN, DN)),
            full((DN, DN)),
            full((1, DN)),
        ],
        out_specs=[
            pl.BlockSpec((_RA, DN), lambda i: (i, 0)),
            pl.BlockSpec((_RA, DN), lambda i: (i, 0)),
            pl.BlockSpec((_RA, DE), lambda i: (i, 0)),
            pl.BlockSpec((_RA, DE), lambda i: (i, 0)),
            pl.BlockSpec((_RA, 1), lambda i: (i, 0)),
        ],
        out_shape=[
            jax.ShapeDtypeStruct((B * N, DN), jnp.float32),
            jax.ShapeDtypeStruct((B * N, DN), jnp.float32),
            jax.ShapeDtypeStruct((B * N, DE), jnp.float32),
            jax.ShapeDtypeStruct((B * N, DE), jnp.float32),
            jax.ShapeDtypeStruct((B * N, 1), jnp.float32),
        ],
    )(nf, c32, W_node, b_node, W1, W2, b_msg)


_RB = 256


def _topk_body(cxr, cyr, czr, cxc, cyc, czc, mc_ref, cc_ref,
               d_ref, i_ref, f_ref, cj_ref):
    b = pl.program_id(0)
    ib = pl.program_id(1)
    xr = cxr[0]
    yr = cyr[0]
    zr = czr[0]
    xc = cxc[0]
    yc = cyc[0]
    zc = czc[0]
    dx = xr - xc
    dy = yr - yc
    dz = zr - zc
    D = jnp.sqrt(dx * dx + dy * dy + dz * dz + 1e-8)
    col = lax.broadcasted_iota(jnp.int32, (1, N), 1)
    row = ib * _RB + lax.broadcasted_iota(jnp.int32, (_RB, 1), 0)
    D = D + jnp.where(row == col, 1e6, 0.0).astype(jnp.float32)
    D = D + (1.0 - mc_ref[0]) * 1e6
    col4 = col * 4 + cc_ref[0]
    big = jnp.int32(4 * N)
    inf = jnp.float32(jnp.inf)
    ds, js, cs = [], [], []
    for _ in range(K):
        m = jnp.min(D, axis=1, keepdims=True)
        cand = jnp.where(D == m, col4, big)
        j4 = jnp.min(cand, axis=1, keepdims=True)
        j = jnp.right_shift(j4, 2)
        ds.append(m)
        js.append(j)
        cs.append(jnp.bitwise_and(j4, 3))
        D = jnp.where(col == j, inf, D)
    d_ref[0] = jnp.concatenate(ds, axis=1)
    i_out = jnp.concatenate(js, axis=1)
    i_ref[0] = i_out
    f_ref[0] = i_out + b * N
    cj_ref[0] = jnp.concatenate(cs, axis=1)


def _topk(centroid, mask_i, c32):
    cr = [centroid[:, :, a:a + 1] for a in range(3)]
    cc = [jnp.transpose(centroid[:, :, a:a + 1], (0, 2, 1)) for a in range(3)]
    mc = mask_i[:, None, :]
    ccol = c32[:, None, :]
    nblk = N // _RB
    rspec = pl.BlockSpec((1, _RB, 1), lambda b, i: (b, i, 0))
    cspec = pl.BlockSpec((1, 1, N), lambda b, i: (b, 0, 0))
    ospec = pl.BlockSpec((1, _RB, K), lambda b, i: (b, i, 0))
    return pl.pallas_call(
        _topk_body,
        grid=(B, nblk),
        in_specs=[rspec, rspec, rspec, cspec, cspec, cspec, cspec, cspec],
        out_specs=[ospec, ospec, ospec, ospec],
        out_shape=[
            jax.ShapeDtypeStruct((B, N, K), jnp.float32),
            jax.ShapeDtypeStruct((B, N, K), jnp.int32),
            jax.ShapeDtypeStruct((B, N, K), jnp.int32),
            jax.ShapeDtypeStruct((B, N, K), jnp.int32),
        ],
    )(cr[0], cr[1], cr[2], cc[0], cc[1], cc[2], mc, ccol)


_CH = 128


def _sc_gather(p2a, p2b, eidx):
    info = plsc.get_sparse_core_info()
    nw = info.num_cores * info.num_subcores
    per_w = E // nw
    nch = per_w // _CH
    mesh = plsc.VectorSubcoreMesh(core_axis_name="c", subcore_axis_name="s")

    @functools.partial(
        pl.kernel,
        mesh=mesh,
        out_type=[
            jax.ShapeDtypeStruct((E, DE), jnp.float32),
            jax.ShapeDtypeStruct((E, DE), jnp.float32),
        ],
        scratch_types=[
            pltpu.VMEM((_CH,), jnp.int32),
            pltpu.VMEM((_CH,), jnp.int32),
            pltpu.VMEM((_CH, DE), jnp.float32),
            pltpu.VMEM((_CH, DE), jnp.float32),
            pltpu.VMEM((_CH, DE), jnp.float32),
            pltpu.VMEM((_CH, DE), jnp.float32),
            pltpu.SemaphoreType.DMA,
            pltpu.SemaphoreType.DMA,
            pltpu.SemaphoreType.DMA,
            pltpu.SemaphoreType.DMA,
        ],
    )
    def k(p2a_hbm, p2b_hbm, eidx_hbm, g0_hbm, g1_hbm,
          idxa, idxb, ra0, ra1, rb0, rb1, sga, sgb, swa, swb):
        wid = lax.axis_index("s") * info.num_cores + lax.axis_index("c")
        base = wid * per_w

        def body(t2, carry):
            offa = base + t2 * 2 * _CH
            offb = offa + _CH
            pltpu.sync_copy(eidx_hbm.at[pl.ds(offa, _CH)], idxa)
            ga0 = pltpu.async_copy(p2a_hbm.at[idxa], ra0, sga)
            ga1 = pltpu.async_copy(p2b_hbm.at[idxa], ra1, sga)
            pltpu.sync_copy(eidx_hbm.at[pl.ds(offb, _CH)], idxb)
            gb0 = pltpu.async_copy(p2a_hbm.at[idxb], rb0, sgb)
            gb1 = pltpu.async_copy(p2b_hbm.at[idxb], rb1, sgb)
            ga0.wait()
            ga1.wait()
            wa0 = pltpu.async_copy(ra0, g0_hbm.at[pl.ds(offa, _CH)], swa)
            wa1 = pltpu.async_copy(ra1, g1_hbm.at[pl.ds(offa, _CH)], swa)
            gb0.wait()
            gb1.wait()
            wb0 = pltpu.async_copy(rb0, g0_hbm.at[pl.ds(offb, _CH)], swb)
            wb1 = pltpu.async_copy(rb1, g1_hbm.at[pl.ds(offb, _CH)], swb)
            wa0.wait()
            wa1.wait()
            wb0.wait()
            wb1.wait()
            return carry

        lax.fori_loop(0, nch // 2, body, 0)

    return k(p2a, p2b, eidx)


_RD = 64
_ED = _RD * K

_SIGMA = 20.0 / 16.0


def _edge_msg_body(d1, cjr, crep, mrep, mi, nh, p1, g0, g1,
                   we_ref, be_ref, w3_ref, cen_ref, sel_ref, selt_ref,
                   eh_ref, mij_ref, no_ref):
    inv = jnp.float32(1.0 / (2.0 * _SIGMA * _SIGMA))
    dd = d1[...] - cen_ref[...]
    rbf = jnp.exp(-(dd * dd) * inv)
    cj = cjr[...]
    same = (crep[...] == cj).astype(jnp.float32)
    we = we_ref[...]
    pre = (jnp.dot(rbf, we[:16, :], preferred_element_type=jnp.float32)
           + same * we[16:17, :] + be_ref[...])
    m_row = mrep[...] * (cj > 0).astype(jnp.float32)
    eh = _sp(pre) * m_row
    eh_ref[...] = eh
    mij_ref[...] = m_row
    sel = sel_ref[...]
    selt = selt_ref[...]
    w3 = w3_ref[...]
    denom = jnp.dot(sel, m_row, preferred_element_type=jnp.float32) + 1e-6
    hi = mi[...]
    for h, gh in ((0, g0), (1, g1)):
        lo = h * DE
        e3 = jnp.dot(eh, w3[:, lo:lo + DE],
                     preferred_element_type=jnp.float32)
        p1b = jnp.dot(selt, p1[:, lo:lo + DE],
                      preferred_element_type=jnp.float32)
        msg = _sp(p1b + gh[...] + e3) * m_row
        msum = jnp.dot(sel, msg, preferred_element_type=jnp.float32)
        no_ref[:, lo:lo + DE] = (nh[:, lo:lo + DE] + msum / denom) * hi


def _edge_msg(d1, cjr, crep, mrep, mi, nh, p1, g0, g1, W_edge, b_edge, W3):
    nblk = (B * N) // _RD
    rs = lambda w: pl.BlockSpec((_RD, w), lambda i: (i, 0))
    es = lambda w: pl.BlockSpec((_ED, w), lambda i: (i, 0))
    full = lambda s: pl.BlockSpec(s, lambda i: (0, 0))
    centers = jnp.linspace(0.0, 20.0, 16).astype(jnp.float32).reshape(1, 16)
    node_of_e = jnp.arange(_ED, dtype=jnp.int32) // K
    sel = (node_of_e[None, :] == jnp.arange(_RD, dtype=jnp.int32)[:, None]
           ).astype(jnp.float32)
    selt = jnp.transpose(sel)
    return pl.pallas_call(
        _edge_msg_body,
        grid=(nblk,),
        in_specs=[
            es(1),
            es(1),
            es(1),
            es(1),
            rs(1),
            rs(DN),
            rs(DN),
            es(DE),
            es(DE),
            full((17, DE)),
            full((1, DE)),
            full((DE, DN)),
            full((1, 16)),
            full((_RD, _ED)),
            full((_ED, _RD)),
        ],
        out_specs=[es(DE), es(1), rs(DN)],
        out_shape=[
            jax.ShapeDtypeStruct((E, DE), jnp.float32),
            jax.ShapeDtypeStruct((E, 1), jnp.float32),
            jax.ShapeDtypeStruct((B * N, DN), jnp.float32),
        ],
    )(d1, cjr, crep, mrep, mi, nh, p1, g0, g1, W_edge, b_edge, W3, centers,
      sel, selt)


def kernel(X, C, W_node, b_node, W_edge, b_edge, W_msg, b_msg):
    c32 = C.astype(jnp.int32)
    nf = X.reshape(B, N, 12).reshape(B * N, 12)
    nf = jnp.pad(nf, ((0, 0), (0, 4)))
    Wn = jnp.pad(W_node, ((0, 4), (0, 0)))
    W1 = W_msg[0:DN]
    W2 = W_msg[DN:2 * DN]
    W3 = W_msg[2 * DN:]

    nh, p1, p2a, p2b, mi_col = _node_feat(
        nf, c32.reshape(B * N, 1), Wn, b_node.reshape(1, DN),
        W1, W2, b_msg.reshape(1, DN))

    centroid = X.mean(axis=2)
    mask_i = mi_col.reshape(B, N)
    d_ij, edge_idx, eidx_flat, cj = _topk(centroid, mask_i, c32)

    g0, g1 = _sc_gather(p2a, p2b, eidx_flat.reshape(E))

    d1 = d_ij.reshape(E, 1)
    cjr = cj.reshape(E, 1)
    crep = jnp.broadcast_to(
        c32.reshape(B * N, 1, 1), (B * N, K, 1)).reshape(E, 1)
    mrep = jnp.broadcast_to(
        mi_col.reshape(B * N, 1, 1), (B * N, K, 1)).reshape(E, 1)

    eh, mij, no = _edge_msg(d1, cjr, crep, mrep, mi_col, nh, p1, g0, g1,
                            W_edge, b_edge.reshape(1, DE), W3)

    return (no.reshape(B, N, DN), eh.reshape(B, N, K, DE), edge_idx,
            mask_i, mij.reshape(B, N, K))


# speedup vs baseline: 9.1144x; 1.0798x over previous
import functools

import jax
import jax.numpy as jnp
from jax import lax
from jax.experimental import pallas as pl
from jax.experimental.pallas import tpu as pltpu
from jax.experimental.pallas import tpu_sc as plsc

B = 4
N = 2048
K = 30
DN = 256
DE = 128
E = B * N * K


_RA = 512


def _sp(x):
    return jnp.maximum(x, 0.0) + jnp.log(1.0 + jnp.exp(-jnp.abs(x)))


def _node_feat_body(nf_ref, c_ref, wn_ref, bn_ref, w1_ref, w2_ref, bm_ref,
                    nh_ref, p1_ref, p2a_ref, p2b_ref, mi_ref):
    mask = (c_ref[...] > 0).astype(jnp.float32)
    nh = _sp(jnp.dot(nf_ref[...], wn_ref[...],
                     preferred_element_type=jnp.float32) + bn_ref[...])
    nh = nh * mask
    nh_ref[...] = nh
    p1_ref[...] = jnp.dot(nh, w1_ref[...],
                          preferred_element_type=jnp.float32) + bm_ref[...]
    p2 = jnp.dot(nh, w2_ref[...], preferred_element_type=jnp.float32)
    p2a_ref[...] = p2[:, :DE]
    p2b_ref[...] = p2[:, DE:]
    mi_ref[...] = mask


def _node_feat(nf, c32, W_node, b_node, W1, W2, b_msg):
    nblk = (B * N) // _RA
    full = lambda s: pl.BlockSpec(s, lambda i: (0, 0))
    return pl.pallas_call(
        _node_feat_body,
        grid=(nblk,),
        in_specs=[
            pl.BlockSpec((_RA, 16), lambda i: (i, 0)),
            pl.BlockSpec((_RA, 1), lambda i: (i, 0)),
            full((16, DN)),
            full((1, DN)),
            full((DN, DN)),
            full((DN, DN)),
            full((1, DN)),
        ],
        out_specs=[
            pl.BlockSpec((_RA, DN), lambda i: (i, 0)),
            pl.BlockSpec((_RA, DN), lambda i: (i, 0)),
            pl.BlockSpec((_RA, DE), lambda i: (i, 0)),
            pl.BlockSpec((_RA, DE), lambda i: (i, 0)),
            pl.BlockSpec((_RA, 1), lambda i: (i, 0)),
        ],
        out_shape=[
            jax.ShapeDtypeStruct((B * N, DN), jnp.float32),
            jax.ShapeDtypeStruct((B * N, DN), jnp.float32),
            jax.ShapeDtypeStruct((B * N, DE), jnp.float32),
            jax.ShapeDtypeStruct((B * N, DE), jnp.float32),
            jax.ShapeDtypeStruct((B * N, 1), jnp.float32),
        ],
    )(nf, c32, W_node, b_node, W1, W2, b_msg)


_RB = 256


def _topk_body(cxr, cyr, czr, cxc, cyc, czc, mc_ref, cc_ref,
               d_ref, i_ref, f_ref, cj_ref):
    b = pl.program_id(0)
    ib = pl.program_id(1)
    xr = cxr[0]
    yr = cyr[0]
    zr = czr[0]
    xc = cxc[0]
    yc = cyc[0]
    zc = czc[0]
    dx = xr - xc
    dy = yr - yc
    dz = zr - zc
    D = jnp.sqrt(dx * dx + dy * dy + dz * dz + 1e-8)
    col = lax.broadcasted_iota(jnp.int32, (1, N), 1)
    row = ib * _RB + lax.broadcasted_iota(jnp.int32, (_RB, 1), 0)
    D = D + jnp.where(row == col, 1e6, 0.0).astype(jnp.float32)
    D = D + (1.0 - mc_ref[0]) * 1e6
    col4 = col * 4 + cc_ref[0]
    big = jnp.int32(4 * N)
    inf = jnp.float32(jnp.inf)
    H = N // 2
    ha = D[:, :H]
    hb = D[:, H:]
    ia = col4[:, :H]
    ib2 = col4[:, H:]
    ale = ha <= hb
    Dp = jnp.where(ale, ha, hb)
    Ds = jnp.where(ale, hb, ha)
    Ip = jnp.where(ale, ia, ib2)
    Is = jnp.where(ale, ib2, ia)
    colp = lax.broadcasted_iota(jnp.int32, (1, H), 1)
    ds, js, cs = [], [], []
    for _ in range(K):
        m = jnp.min(Dp, axis=1, keepdims=True)
        cand = jnp.where(Dp == m, Ip, big)
        j4 = jnp.min(cand, axis=1, keepdims=True)
        j = jnp.right_shift(j4, 2)
        ds.append(m)
        js.append(j)
        cs.append(jnp.bitwise_and(j4, 3))
        sel = colp == jnp.bitwise_and(j, jnp.int32(H - 1))
        Dp = jnp.where(sel, Ds, Dp)
        Ip = jnp.where(sel, Is, Ip)
        Ds = jnp.where(sel, inf, Ds)
    d_ref[0] = jnp.concatenate(ds, axis=1)
    i_out = jnp.concatenate(js, axis=1)
    i_ref[0] = i_out
    f_ref[0] = i_out + b * N
    cj_ref[0] = jnp.concatenate(cs, axis=1)


def _topk(centroid, mask_i, c32):
    cr = [centroid[:, :, a:a + 1] for a in range(3)]
    cc = [jnp.transpose(centroid[:, :, a:a + 1], (0, 2, 1)) for a in range(3)]
    mc = mask_i[:, None, :]
    ccol = c32[:, None, :]
    nblk = N // _RB
    rspec = pl.BlockSpec((1, _RB, 1), lambda b, i: (b, i, 0))
    cspec = pl.BlockSpec((1, 1, N), lambda b, i: (b, 0, 0))
    ospec = pl.BlockSpec((1, _RB, K), lambda b, i: (b, i, 0))
    return pl.pallas_call(
        _topk_body,
        grid=(B, nblk),
        in_specs=[rspec, rspec, rspec, cspec, cspec, cspec, cspec, cspec],
        out_specs=[ospec, ospec, ospec, ospec],
        out_shape=[
            jax.ShapeDtypeStruct((B, N, K), jnp.float32),
            jax.ShapeDtypeStruct((B, N, K), jnp.int32),
            jax.ShapeDtypeStruct((B, N, K), jnp.int32),
            jax.ShapeDtypeStruct((B, N, K), jnp.int32),
        ],
    )(cr[0], cr[1], cr[2], cc[0], cc[1], cc[2], mc, ccol)


_CH = 128


def _sc_gather(p2a, p2b, eidx):
    info = plsc.get_sparse_core_info()
    nw = info.num_cores * info.num_subcores
    per_w = E // nw
    nch = per_w // _CH
    mesh = plsc.VectorSubcoreMesh(core_axis_name="c", subcore_axis_name="s")

    @functools.partial(
        pl.kernel,
        mesh=mesh,
        out_type=[
            jax.ShapeDtypeStruct((E, DE), jnp.float32),
            jax.ShapeDtypeStruct((E, DE), jnp.float32),
        ],
        scratch_types=[
            pltpu.VMEM((_CH,), jnp.int32),
            pltpu.VMEM((_CH,), jnp.int32),
            pltpu.VMEM((_CH, DE), jnp.float32),
            pltpu.VMEM((_CH, DE), jnp.float32),
            pltpu.VMEM((_CH, DE), jnp.float32),
            pltpu.VMEM((_CH, DE), jnp.float32),
            pltpu.SemaphoreType.DMA,
            pltpu.SemaphoreType.DMA,
            pltpu.SemaphoreType.DMA,
            pltpu.SemaphoreType.DMA,
        ],
    )
    def k(p2a_hbm, p2b_hbm, eidx_hbm, g0_hbm, g1_hbm,
          idxa, idxb, ra0, ra1, rb0, rb1, sga, sgb, swa, swb):
        wid = lax.axis_index("s") * info.num_cores + lax.axis_index("c")
        base = wid * per_w

        def body(t2, carry):
            offa = base + t2 * 2 * _CH
            offb = offa + _CH
            pltpu.sync_copy(eidx_hbm.at[pl.ds(offa, _CH)], idxa)
            ga0 = pltpu.async_copy(p2a_hbm.at[idxa], ra0, sga)
            ga1 = pltpu.async_copy(p2b_hbm.at[idxa], ra1, sga)
            pltpu.sync_copy(eidx_hbm.at[pl.ds(offb, _CH)], idxb)
            gb0 = pltpu.async_copy(p2a_hbm.at[idxb], rb0, sgb)
            gb1 = pltpu.async_copy(p2b_hbm.at[idxb], rb1, sgb)
            ga0.wait()
            ga1.wait()
            wa0 = pltpu.async_copy(ra0, g0_hbm.at[pl.ds(offa, _CH)], swa)
            wa1 = pltpu.async_copy(ra1, g1_hbm.at[pl.ds(offa, _CH)], swa)
            gb0.wait()
            gb1.wait()
            wb0 = pltpu.async_copy(rb0, g0_hbm.at[pl.ds(offb, _CH)], swb)
            wb1 = pltpu.async_copy(rb1, g1_hbm.at[pl.ds(offb, _CH)], swb)
            wa0.wait()
            wa1.wait()
            wb0.wait()
            wb1.wait()
            return carry

        lax.fori_loop(0, nch // 2, body, 0)

    return k(p2a, p2b, eidx)


_RD = 64
_ED = _RD * K

_SIGMA = 20.0 / 16.0


def _edge_msg_body(d1, cjr, crep, mrep, mi, nh, p1, g0, g1,
                   we_ref, be_ref, w3_ref, cen_ref, sel_ref, selt_ref,
                   eh_ref, mij_ref, no_ref):
    inv = jnp.float32(1.0 / (2.0 * _SIGMA * _SIGMA))
    dd = d1[...] - cen_ref[...]
    rbf = jnp.exp(-(dd * dd) * inv)
    cj = cjr[...]
    same = (crep[...] == cj).astype(jnp.float32)
    we = we_ref[...]
    pre = (jnp.dot(rbf, we[:16, :], preferred_element_type=jnp.float32)
           + same * we[16:17, :] + be_ref[...])
    m_row = mrep[...] * (cj > 0).astype(jnp.float32)
    eh = _sp(pre) * m_row
    eh_ref[...] = eh
    mij_ref[...] = m_row
    sel = sel_ref[...]
    selt = selt_ref[...]
    w3 = w3_ref[...]
    denom = jnp.dot(sel, m_row, preferred_element_type=jnp.float32) + 1e-6
    hi = mi[...]
    for h, gh in ((0, g0), (1, g1)):
        lo = h * DE
        e3 = jnp.dot(eh, w3[:, lo:lo + DE],
                     preferred_element_type=jnp.float32)
        p1b = jnp.dot(selt, p1[:, lo:lo + DE],
                      preferred_element_type=jnp.float32)
        msg = _sp(p1b + gh[...] + e3) * m_row
        msum = jnp.dot(sel, msg, preferred_element_type=jnp.float32)
        no_ref[:, lo:lo + DE] = (nh[:, lo:lo + DE] + msum / denom) * hi


def _edge_msg(d1, cjr, crep, mrep, mi, nh, p1, g0, g1, W_edge, b_edge, W3):
    nblk = (B * N) // _RD
    rs = lambda w: pl.BlockSpec((_RD, w), lambda i: (i, 0))
    es = lambda w: pl.BlockSpec((_ED, w), lambda i: (i, 0))
    full = lambda s: pl.BlockSpec(s, lambda i: (0, 0))
    centers = jnp.linspace(0.0, 20.0, 16).astype(jnp.float32).reshape(1, 16)
    node_of_e = jnp.arange(_ED, dtype=jnp.int32) // K
    sel = (node_of_e[None, :] == jnp.arange(_RD, dtype=jnp.int32)[:, None]
           ).astype(jnp.float32)
    selt = jnp.transpose(sel)
    return pl.pallas_call(
        _edge_msg_body,
        grid=(nblk,),
        in_specs=[
            es(1),
            es(1),
            es(1),
            es(1),
            rs(1),
            rs(DN),
            rs(DN),
            es(DE),
            es(DE),
            full((17, DE)),
            full((1, DE)),
            full((DE, DN)),
            full((1, 16)),
            full((_RD, _ED)),
            full((_ED, _RD)),
        ],
        out_specs=[es(DE), es(1), rs(DN)],
        out_shape=[
            jax.ShapeDtypeStruct((E, DE), jnp.float32),
            jax.ShapeDtypeStruct((E, 1), jnp.float32),
            jax.ShapeDtypeStruct((B * N, DN), jnp.float32),
        ],
    )(d1, cjr, crep, mrep, mi, nh, p1, g0, g1, W_edge, b_edge, W3, centers,
      sel, selt)


def kernel(X, C, W_node, b_node, W_edge, b_edge, W_msg, b_msg):
    c32 = C.astype(jnp.int32)
    nf = X.reshape(B, N, 12).reshape(B * N, 12)
    nf = jnp.pad(nf, ((0, 0), (0, 4)))
    Wn = jnp.pad(W_node, ((0, 4), (0, 0)))
    W1 = W_msg[0:DN]
    W2 = W_msg[DN:2 * DN]
    W3 = W_msg[2 * DN:]

    nh, p1, p2a, p2b, mi_col = _node_feat(
        nf, c32.reshape(B * N, 1), Wn, b_node.reshape(1, DN),
        W1, W2, b_msg.reshape(1, DN))

    centroid = X.mean(axis=2)
    mask_i = mi_col.reshape(B, N)
    d_ij, edge_idx, eidx_flat, cj = _topk(centroid, mask_i, c32)

    g0, g1 = _sc_gather(p2a, p2b, eidx_flat.reshape(E))

    d1 = d_ij.reshape(E, 1)
    cjr = cj.reshape(E, 1)
    crep = jnp.broadcast_to(
        c32.reshape(B * N, 1, 1), (B * N, K, 1)).reshape(E, 1)
    mrep = jnp.broadcast_to(
        mi_col.reshape(B * N, 1, 1), (B * N, K, 1)).reshape(E, 1)

    eh, mij, no = _edge_msg(d1, cjr, crep, mrep, mi_col, nh, p1, g0, g1,
                            W_edge, b_edge.reshape(1, DE), W3)

    return (no.reshape(B, N, DN), eh.reshape(B, N, K, DE), edge_idx,
            mask_i, mij.reshape(B, N, K))
